# baseline (speedup 1.0000x reference)
import functools

import jax
import jax.numpy as jnp
from jax import lax
from jax.experimental import pallas as pl
from jax.experimental.pallas import tpu as pltpu
from jax.experimental.pallas import tpu_sc as plsc

_Q = 0.7
_K = 0.5
_C2 = (1.0 - _K ** _Q) / _Q


def _sc_gather(table, idx):
    B = idx.shape[0]
    info = plsc.get_sparse_core_info()
    nw = info.num_cores * info.num_subcores
    bpw = B // nw
    mesh = plsc.VectorSubcoreMesh(core_axis_name="c", subcore_axis_name="s")

    @functools.partial(
        pl.kernel,
        mesh=mesh,
        out_type=jax.ShapeDtypeStruct((B,), jnp.float32),
        scratch_types=[
            pltpu.VMEM((bpw,), jnp.int32),
            pltpu.VMEM((bpw,), jnp.float32),
            pltpu.SemaphoreType.DMA,
        ],
    )
    def k(table_hbm, idx_hbm, out_hbm, idx_v, rows_v, sem):
        wid = lax.axis_index("s") * info.num_cores + lax.axis_index("c")
        base = wid * bpw
        pltpu.sync_copy(idx_hbm.at[pl.ds(base, bpw)], idx_v)
        pltpu.async_copy(table_hbm.at[idx_v], rows_v, sem).wait()
        pltpu.sync_copy(rows_v, out_hbm.at[pl.ds(base, bpw)])

    return k(table, idx)


def _g_of(x, t2d):
    m = jnp.max(x, axis=1, keepdims=True)
    s = jnp.sum(jnp.exp(x - m), axis=1, keepdims=True)
    cols = lax.broadcasted_iota(jnp.int32, x.shape, 1)
    lt = jnp.sum(jnp.where(cols == t2d, x, 0.0), axis=1, keepdims=True)
    log_yg = lt - m - jnp.log(s)
    return (1.0 - jnp.exp(_Q * log_yg)) / _Q - _C2


_R = 512
_NBUF = 3


def _tc_dense_body(nsteps, x_hbm, t_ref, g_ref, buf, sems):
    i = pl.program_id(0)
    R = _R

    def start(step):
        slot = lax.rem(step, _NBUF)
        pltpu.make_async_copy(
            x_hbm.at[pl.ds(step * R, R), :], buf.at[slot], sems.at[slot]
        ).start()

    @pl.when(i == 0)
    def _():
        for j in range(_NBUF - 1):
            start(j)

    @pl.when(i + _NBUF - 1 < nsteps)
    def _():
        start(i + _NBUF - 1)

    slot = lax.rem(i, _NBUF)
    pltpu.make_async_copy(
        x_hbm.at[pl.ds(i * R, R), :], buf.at[slot], sems.at[slot]
    ).wait()
    g_ref[:, :] = _g_of(buf[slot], t_ref[:, :])


def _tc_dense(logits, targets2d):
    B, C = logits.shape
    nsteps = B // _R
    return pl.pallas_call(
        functools.partial(_tc_dense_body, nsteps),
        grid=(nsteps,),
        in_specs=[
            pl.BlockSpec(memory_space=pl.ANY),
            pl.BlockSpec((_R, 1), lambda i: (i, 0)),
        ],
        out_specs=pl.BlockSpec((_R, 1), lambda i: (i, 0)),
        out_shape=jax.ShapeDtypeStruct((B, 1), jnp.float32),
        scratch_shapes=[
            pltpu.VMEM((_NBUF, _R, C), jnp.float32),
            pltpu.SemaphoreType.DMA((_NBUF,)),
        ],
    )(logits, targets2d)


def _combine_body(inv_b, g_ref, w_ref, o_ref):
    o_ref[0, 0] = jnp.sum(g_ref[:, :] * w_ref[:, :]) * inv_b


def _combine(g2d, w2d, B):
    return pl.pallas_call(
        functools.partial(_combine_body, 1.0 / B),
        out_specs=pl.BlockSpec(memory_space=pltpu.SMEM),
        out_shape=jax.ShapeDtypeStruct((1, 1), jnp.float32),
    )(g2d, w2d)


@jax.jit
def kernel(logits, targets, indexes, weight):
    B = logits.shape[0]
    w = _sc_gather(weight.reshape(-1), indexes)
    g = _tc_dense(logits, targets.reshape(B, 1))
    out = _combine(g, w.reshape(B, 1), B)
    return out[0, 0]


# speedup vs baseline: 1.7971x; 1.7971x over previous
import functools

import jax
import jax.numpy as jnp
from jax import lax
from jax.experimental import pallas as pl
from jax.experimental.pallas import tpu as pltpu
from jax.experimental.pallas import tpu_sc as plsc

_Q = 0.7
_K = 0.5
_C2 = (1.0 - _K ** _Q) / _Q


def _sc_gather(table, idx):
    B = idx.shape[0]
    info = plsc.get_sparse_core_info()
    nw = info.num_cores * info.num_subcores
    bpw = B // nw
    mesh = plsc.VectorSubcoreMesh(core_axis_name="c", subcore_axis_name="s")

    @functools.partial(
        pl.kernel,
        mesh=mesh,
        out_type=jax.ShapeDtypeStruct((B,), jnp.float32),
        scratch_types=[
            pltpu.VMEM((bpw,), jnp.int32),
            pltpu.VMEM((bpw,), jnp.float32),
            pltpu.SemaphoreType.DMA,
        ],
    )
    def k(table_hbm, idx_hbm, out_hbm, idx_v, rows_v, sem):
        wid = lax.axis_index("s") * info.num_cores + lax.axis_index("c")
        base = wid * bpw
        pltpu.sync_copy(idx_hbm.at[pl.ds(base, bpw)], idx_v)
        pltpu.async_copy(table_hbm.at[idx_v], rows_v, sem).wait()
        pltpu.sync_copy(rows_v, out_hbm.at[pl.ds(base, bpw)])

    return k(table, idx)


def _tc_dense_body(x_ref, t_ref, g_ref):
    x = x_ref[:, :]
    m = jnp.max(x, axis=0, keepdims=True)
    s = jnp.sum(jnp.exp(x - m), axis=0, keepdims=True)
    rows = lax.broadcasted_iota(jnp.int32, x.shape, 0)
    lt = jnp.sum(jnp.where(rows == t_ref[:, :], x, 0.0), axis=0, keepdims=True)
    log_yg = lt - m - jnp.log(s)
    g_ref[:, :] = (1.0 - jnp.exp(_Q * log_yg)) / _Q - _C2


def _tc_dense(logits_t, targets_row):
    C, B = logits_t.shape
    R = 512
    nsteps = B // R
    return pl.pallas_call(
        _tc_dense_body,
        grid=(nsteps,),
        in_specs=[
            pl.BlockSpec((C, R), lambda i: (0, i)),
            pl.BlockSpec((1, R), lambda i: (0, i)),
        ],
        out_specs=pl.BlockSpec((1, R), lambda i: (0, i)),
        out_shape=jax.ShapeDtypeStruct((1, B), jnp.float32),
    )(logits_t, targets_row)


def _combine_body(inv_b, g_ref, w_ref, o_ref):
    o_ref[0, 0] = jnp.sum(g_ref[:, :] * w_ref[:, :]) * inv_b


def _combine(g2d, w2d, B):
    return pl.pallas_call(
        functools.partial(_combine_body, 1.0 / B),
        out_specs=pl.BlockSpec(memory_space=pltpu.SMEM),
        out_shape=jax.ShapeDtypeStruct((1, 1), jnp.float32),
    )(g2d, w2d)


@jax.jit
def kernel(logits, targets, indexes, weight):
    B = logits.shape[0]
    w = _sc_gather(weight.reshape(-1), indexes)
    lgt = pltpu.with_memory_space_constraint(logits.T, pltpu.MemorySpace.HBM)
    g = _tc_dense(lgt, targets.reshape(1, B))
    out = _combine(g, w.reshape(1, B), B)
    return out[0, 0]


# speedup vs baseline: 1.8039x; 1.0038x over previous
import functools

import jax
import jax.numpy as jnp
from jax import lax
from jax.experimental import pallas as pl
from jax.experimental.pallas import tpu as pltpu
from jax.experimental.pallas import tpu_sc as plsc

_Q = 0.7
_K = 0.5
_C2 = (1.0 - _K ** _Q) / _Q


def _sc_gather(table, idx):
    B = idx.shape[0]
    info = plsc.get_sparse_core_info()
    nw = info.num_cores * info.num_subcores
    bpw = B // nw
    mesh = plsc.VectorSubcoreMesh(core_axis_name="c", subcore_axis_name="s")

    @functools.partial(
        pl.kernel,
        mesh=mesh,
        out_type=jax.ShapeDtypeStruct((B,), jnp.float32),
        scratch_types=[
            pltpu.VMEM((bpw,), jnp.int32),
            pltpu.VMEM((bpw,), jnp.float32),
            pltpu.SemaphoreType.DMA,
        ],
        compiler_params=pltpu.CompilerParams(skip_device_barrier=True),
    )
    def k(table_hbm, idx_hbm, out_hbm, idx_v, rows_v, sem):
        wid = lax.axis_index("s") * info.num_cores + lax.axis_index("c")
        base = wid * bpw
        pltpu.sync_copy(idx_hbm.at[pl.ds(base, bpw)], idx_v)
        pltpu.async_copy(table_hbm.at[idx_v], rows_v, sem).wait()
        pltpu.sync_copy(rows_v, out_hbm.at[pl.ds(base, bpw)])

    return k(table, idx)


def _tc_dense_body(x_ref, t_ref, g_ref):
    x = x_ref[:, :]
    m = jnp.max(x, axis=0, keepdims=True)
    s = jnp.sum(jnp.exp(x - m), axis=0, keepdims=True)
    rows = lax.broadcasted_iota(jnp.int32, x.shape, 0)
    lt = jnp.sum(jnp.where(rows == t_ref[:, :], x, 0.0), axis=0, keepdims=True)
    log_yg = lt - m - jnp.log(s)
    g_ref[:, :] = (1.0 - jnp.exp(_Q * log_yg)) / _Q - _C2


def _tc_dense(logits_t, targets_row):
    C, B = logits_t.shape
    R = 512
    nsteps = B // R
    return pl.pallas_call(
        _tc_dense_body,
        grid=(nsteps,),
        in_specs=[
            pl.BlockSpec((C, R), lambda i: (0, i)),
            pl.BlockSpec((1, R), lambda i: (0, i)),
        ],
        out_specs=pl.BlockSpec((1, R), lambda i: (0, i)),
        out_shape=jax.ShapeDtypeStruct((1, B), jnp.float32),
    )(logits_t, targets_row)


def _combine_body(inv_b, g_ref, w_ref, o_ref):
    o_ref[0, 0] = jnp.sum(g_ref[:, :] * w_ref[:, :]) * inv_b


def _combine(g2d, w2d, B):
    return pl.pallas_call(
        functools.partial(_combine_body, 1.0 / B),
        out_specs=pl.BlockSpec(memory_space=pltpu.SMEM),
        out_shape=jax.ShapeDtypeStruct((1, 1), jnp.float32),
    )(g2d, w2d)


@jax.jit
def kernel(logits, targets, indexes, weight):
    B = logits.shape[0]
    w = _sc_gather(weight.reshape(-1), indexes)
    lgt = pltpu.with_memory_space_constraint(logits.T, pltpu.MemorySpace.HBM)
    g = _tc_dense(lgt, targets.reshape(1, B))
    out = _combine(g, w.reshape(1, B), B)
    return out[0, 0]


# speedup vs baseline: 1.8845x; 1.0447x over previous
import functools

import jax
import jax.numpy as jnp
from jax import lax
from jax.experimental import pallas as pl
from jax.experimental.pallas import tpu as pltpu
from jax.experimental.pallas import tpu_sc as plsc

_Q = 0.7
_K = 0.5
_C2 = (1.0 - _K ** _Q) / _Q


def _sc_gather(table, idx):
    B = idx.shape[0]
    info = plsc.get_sparse_core_info()
    nw = info.num_cores * info.num_subcores
    bpw = B // nw
    mesh = plsc.VectorSubcoreMesh(core_axis_name="c", subcore_axis_name="s")

    @functools.partial(
        pl.kernel,
        mesh=mesh,
        out_type=jax.ShapeDtypeStruct((B,), jnp.float32),
        scratch_types=[
            pltpu.VMEM((bpw,), jnp.int32),
            pltpu.VMEM((bpw,), jnp.float32),
            pltpu.SemaphoreType.DMA,
        ],
        compiler_params=pltpu.CompilerParams(skip_device_barrier=True),
    )
    def k(table_hbm, idx_hbm, out_hbm, idx_v, rows_v, sem):
        wid = lax.axis_index("s") * info.num_cores + lax.axis_index("c")
        base = wid * bpw
        pltpu.sync_copy(idx_hbm.at[pl.ds(base, bpw)], idx_v)
        pltpu.async_copy(table_hbm.at[idx_v], rows_v, sem).wait()
        pltpu.sync_copy(rows_v, out_hbm.at[pl.ds(base, bpw)])

    return k(table, idx)


def _g_of(x, t_row):
    m = jnp.max(x, axis=0, keepdims=True)
    s = jnp.sum(jnp.exp(x - m), axis=0, keepdims=True)
    rows = lax.broadcasted_iota(jnp.int32, x.shape, 0)
    lt = jnp.sum(jnp.where(rows == t_row, x, 0.0), axis=0, keepdims=True)
    log_yg = lt - m - jnp.log(s)
    return (1.0 - jnp.exp(_Q * log_yg)) / _Q - _C2


def _tc_dense_body(x1_ref, x2_ref, t1_ref, t2_ref, g1_ref, g2_ref):
    g1_ref[:, :] = _g_of(x1_ref[:, :], t1_ref[:, :])
    g2_ref[:, :] = _g_of(x2_ref[:, :], t2_ref[:, :])


def _tc_dense(logits_t, targets_row):
    C, B = logits_t.shape
    R = 512
    nsteps = B // R // 2
    return pl.pallas_call(
        _tc_dense_body,
        grid=(nsteps,),
        in_specs=[
            pl.BlockSpec((C, R), lambda i: (0, i)),
            pl.BlockSpec((C, R), lambda i: (0, i + 4)),
            pl.BlockSpec((1, R), lambda i: (0, i)),
            pl.BlockSpec((1, R), lambda i: (0, i + 4)),
        ],
        out_specs=[
            pl.BlockSpec((1, R), lambda i: (0, i)),
            pl.BlockSpec((1, R), lambda i: (0, i)),
        ],
        out_shape=[
            jax.ShapeDtypeStruct((1, B // 2), jnp.float32),
            jax.ShapeDtypeStruct((1, B // 2), jnp.float32),
        ],
    )(logits_t, logits_t, targets_row, targets_row)


def _combine_body(inv_b, g1_ref, g2_ref, w_ref, o_ref):
    h = w_ref.shape[1] // 2
    tot = (jnp.sum(g1_ref[:, :] * w_ref[:, :h])
           + jnp.sum(g2_ref[:, :] * w_ref[:, h:]))
    o_ref[0, 0] = tot * inv_b


def _combine(g1, g2, w2d, B):
    return pl.pallas_call(
        functools.partial(_combine_body, 1.0 / B),
        out_specs=pl.BlockSpec(memory_space=pltpu.SMEM),
        out_shape=jax.ShapeDtypeStruct((1, 1), jnp.float32),
    )(g1, g2, w2d)


@jax.jit
def kernel(logits, targets, indexes, weight):
    B = logits.shape[0]
    w = _sc_gather(weight.reshape(-1), indexes)
    lgt = pltpu.with_memory_space_constraint(logits.T, pltpu.MemorySpace.HBM)
    g1, g2 = _tc_dense(lgt, targets.reshape(1, B))
    out = _combine(g1, g2, w.reshape(1, B), B)
    return out[0, 0]


# speedup vs baseline: 1.8861x; 1.0008x over previous
import functools

import jax
import jax.numpy as jnp
from jax import lax
from jax.experimental import pallas as pl
from jax.experimental.pallas import tpu as pltpu
from jax.experimental.pallas import tpu_sc as plsc

_Q = 0.7
_K = 0.5
_C2 = (1.0 - _K ** _Q) / _Q


def _sc_gather(table, idx):
    B = idx.shape[0]
    info = plsc.get_sparse_core_info()
    nw = info.num_cores * info.num_subcores
    bpw = B // nw
    mesh = plsc.VectorSubcoreMesh(core_axis_name="c", subcore_axis_name="s")

    @functools.partial(
        pl.kernel,
        mesh=mesh,
        out_type=jax.ShapeDtypeStruct((B,), jnp.float32),
        scratch_types=[
            pltpu.VMEM((bpw,), jnp.int32),
            pltpu.VMEM((bpw,), jnp.float32),
            pltpu.SemaphoreType.DMA,
        ],
        compiler_params=pltpu.CompilerParams(skip_device_barrier=True),
    )
    def k(table_hbm, idx_hbm, out_hbm, idx_v, rows_v, sem):
        wid = lax.axis_index("s") * info.num_cores + lax.axis_index("c")
        base = wid * bpw
        pltpu.sync_copy(idx_hbm.at[pl.ds(base, bpw)], idx_v)
        pltpu.async_copy(table_hbm.at[idx_v], rows_v, sem).wait()
        pltpu.sync_copy(rows_v, out_hbm.at[pl.ds(base, bpw)])

    return k(table, idx)


def _g_of(x, t_row):
    m = jnp.max(x, axis=0, keepdims=True)
    s = jnp.sum(jnp.exp(x - m), axis=0, keepdims=True)
    rows = lax.broadcasted_iota(jnp.int32, x.shape, 0)
    lt = jnp.sum(jnp.where(rows == t_row, x, 0.0), axis=0, keepdims=True)
    log_yg = lt - m - jnp.log(s)
    return (1.0 - jnp.exp(_Q * log_yg)) / _Q - _C2


def _tc_dense_body(x1_ref, x2_ref, x3_ref, x4_ref,
                   t1_ref, t2_ref, t3_ref, t4_ref,
                   g1_ref, g2_ref, g3_ref, g4_ref):
    g1_ref[:, :] = _g_of(x1_ref[:, :], t1_ref[:, :])
    g2_ref[:, :] = _g_of(x2_ref[:, :], t2_ref[:, :])
    g3_ref[:, :] = _g_of(x3_ref[:, :], t3_ref[:, :])
    g4_ref[:, :] = _g_of(x4_ref[:, :], t4_ref[:, :])


def _tc_dense(logits_t, targets_row):
    C, B = logits_t.shape
    R = 512
    nsteps = B // R // 4
    x_specs = [pl.BlockSpec((C, R), functools.partial(
        lambda q, i: (0, i + q * nsteps), q)) for q in range(4)]
    t_specs = [pl.BlockSpec((1, R), functools.partial(
        lambda q, i: (0, i + q * nsteps), q)) for q in range(4)]
    return pl.pallas_call(
        _tc_dense_body,
        grid=(nsteps,),
        in_specs=x_specs + t_specs,
        out_specs=[pl.BlockSpec((1, R), lambda i: (0, i))] * 4,
        out_shape=[jax.ShapeDtypeStruct((1, B // 4), jnp.float32)] * 4,
    )(logits_t, logits_t, logits_t, logits_t,
      targets_row, targets_row, targets_row, targets_row)


def _combine_body(inv_b, g1_ref, g2_ref, g3_ref, g4_ref, w_ref, o_ref):
    q = w_ref.shape[1] // 4
    tot = (jnp.sum(g1_ref[:, :] * w_ref[:, 0 * q:1 * q])
           + jnp.sum(g2_ref[:, :] * w_ref[:, 1 * q:2 * q])
           + jnp.sum(g3_ref[:, :] * w_ref[:, 2 * q:3 * q])
           + jnp.sum(g4_ref[:, :] * w_ref[:, 3 * q:4 * q]))
    o_ref[0, 0] = tot * inv_b


def _combine(gs, w2d, B):
    return pl.pallas_call(
        functools.partial(_combine_body, 1.0 / B),
        out_specs=pl.BlockSpec(memory_space=pltpu.SMEM),
        out_shape=jax.ShapeDtypeStruct((1, 1), jnp.float32),
    )(*gs, w2d)


@jax.jit
def kernel(logits, targets, indexes, weight):
    B = logits.shape[0]
    w = _sc_gather(weight.reshape(-1), indexes)
    lgt = pltpu.with_memory_space_constraint(logits.T, pltpu.MemorySpace.HBM)
    gs = _tc_dense(lgt, targets.reshape(1, B))
    out = _combine(gs, w.reshape(1, B), B)
    return out[0, 0]
